# baseline (speedup 1.0000x reference)
import functools

import jax
import jax.numpy as jnp
from jax.experimental import pallas as pl
from jax.experimental.pallas import tpu as pltpu

KERNEL_NUM = 21
SIGMA = 0.1
EXACT_SIGMA = 0.001
EPS = 1e-8

_N = KERNEL_NUM - 1
_MUS = [1.0 / _N + 2.0 * i / _N - 1.0 for i in range(_N)] + [1.0]
_SIGS = [SIGMA] * _N + [EXACT_SIGMA]


def _km_kernel(idx_ref, emb_hbm, o1_ref, o2_ref, rows, sem, *, nb, ql, dl, d):
    f1 = 2 * (ql + dl)
    total = nb * f1

    unroll = 16
    n_chunks = total // unroll

    def issue_chunk(c, _):
        base = c * unroll
        for u in range(unroll):
            n = base + u
            t = idx_ref[0, 0, n]
            pltpu.make_async_copy(
                emb_hbm.at[pl.ds(t, 1), :], rows.at[pl.ds(n, 1), :], sem
            ).start()
        return 0

    jax.lax.fori_loop(0, n_chunks, issue_chunk, 0)
    pltpu.make_async_copy(
        emb_hbm.at[pl.ds(0, total), :], rows.at[pl.ds(0, total), :], sem
    ).wait()

    r = rows[...]
    ss = jnp.sum(r * r, axis=1, keepdims=True)
    rows[...] = r * jax.lax.rsqrt(jnp.maximum(ss, EPS * EPS))

    mm_parts = []
    for pair in range(2):
        for jb in range(nb):
            base = jb * f1 + pair * (ql + dl)
            qn = rows[pl.ds(base, ql), :]
            dn = rows[pl.ds(base + ql, dl), :]
            mm = jax.lax.dot_general(
                qn, dn, (((1,), (1,)), ((), ())),
                preferred_element_type=jnp.float32,
            )
            mm_parts.append(mm)
    mm_all = jnp.concatenate(mm_parts, axis=0)

    s_cols = []
    for k in range(KERNEL_NUM):
        c = -0.5 / (_SIGS[k] * _SIGS[k])
        t = mm_all - _MUS[k]
        e = jnp.exp(c * t * t)
        s_cols.append(jnp.sum(e, axis=1, keepdims=True))
    s = jnp.concatenate(s_cols, axis=1)
    l1p = jnp.log1p(s)

    g = 2 * nb
    row_id = jax.lax.broadcasted_iota(jnp.int32, (g * ql, g), 0) // ql
    col_id = jax.lax.broadcasted_iota(jnp.int32, (g * ql, g), 1)
    gmat = (row_id == col_id).astype(jnp.float32)
    km = jax.lax.dot_general(
        gmat, l1p, (((0,), (0,)), ((), ())),
        preferred_element_type=jnp.float32,
        precision=jax.lax.Precision.HIGHEST,
    )
    o1_ref[0] = km[0:nb, :]
    o2_ref[0] = km[nb:2 * nb, :]


def _mlp_kernel(km1_ref, km2_ref, w0_ref, b0_ref, w1_ref, b1_ref,
                w2_ref, b2_ref, o_ref):
    def head(x):
        h = jax.lax.dot_general(
            x, w0_ref[...], (((1,), (1,)), ((), ())),
            preferred_element_type=jnp.float32,
        ) + b0_ref[...]
        h = jnp.maximum(h, 0.0)
        h = jax.lax.dot_general(
            h, w1_ref[...], (((1,), (1,)), ((), ())),
            preferred_element_type=jnp.float32,
        ) + b1_ref[...]
        h = jnp.maximum(h, 0.0)
        l = jax.lax.dot_general(
            h, w2_ref[...], (((1,), (1,)), ((), ())),
            preferred_element_type=jnp.float32,
        )
        return l[:, 0:1] + b2_ref[0, 0]

    diff = head(km1_ref[...]) - head(km2_ref[...])
    o_ref[...] = 1.0 / (1.0 + jnp.exp(-diff))


def kernel(query_1, document_1, query_2, document_2,
           emb_weight, W0, b0, W1, b1, W2, b2):
    B, QL = query_1.shape
    _, DL = document_1.shape
    V, D = emb_weight.shape

    nb = 8
    while B % nb:
        nb //= 2
    S = B // nb
    f1 = 2 * (QL + DL)

    idx = jnp.concatenate(
        [query_1, document_1, query_2, document_2], axis=1
    ).astype(jnp.int32)
    idx3 = idx.reshape(S, 1, nb * f1)


    km_body = functools.partial(_km_kernel, nb=nb, ql=QL, dl=DL, d=D)
    km1, km2 = pl.pallas_call(
        km_body,
        grid=(S,),
        in_specs=[
            pl.BlockSpec((1, 1, nb * f1), lambda i: (i, 0, 0),
                         memory_space=pltpu.SMEM),
            pl.BlockSpec(memory_space=pl.ANY),
        ],
        out_specs=[
            pl.BlockSpec((1, nb, KERNEL_NUM), lambda i: (i, 0, 0)),
            pl.BlockSpec((1, nb, KERNEL_NUM), lambda i: (i, 0, 0)),
        ],
        out_shape=[
            jax.ShapeDtypeStruct((S, nb, KERNEL_NUM), jnp.float32),
            jax.ShapeDtypeStruct((S, nb, KERNEL_NUM), jnp.float32),
        ],
        scratch_shapes=[
            pltpu.VMEM((nb * f1, D), jnp.float32),
            pltpu.SemaphoreType.DMA,
        ],
        compiler_params=pltpu.CompilerParams(
            dimension_semantics=("arbitrary",),
            disable_bounds_checks=True,
        ),
    )(idx3, emb_weight)

    out = pl.pallas_call(
        _mlp_kernel,
        out_shape=jax.ShapeDtypeStruct((B, 1), jnp.float32),
    )(
        km1.reshape(B, KERNEL_NUM),
        km2.reshape(B, KERNEL_NUM),
        W0, b0.reshape(1, -1), W1, b1.reshape(1, -1),
        jnp.pad(W2, ((0, 7), (0, 0))), b2.reshape(1, -1),
    )
    return out


# speedup vs baseline: 1.0257x; 1.0257x over previous
import functools

import jax
import jax.numpy as jnp
from jax.experimental import pallas as pl
from jax.experimental.pallas import tpu as pltpu

KERNEL_NUM = 21
SIGMA = 0.1
EXACT_SIGMA = 0.001
EPS = 1e-8

_N = KERNEL_NUM - 1
_MUS = [1.0 / _N + 2.0 * i / _N - 1.0 for i in range(_N)] + [1.0]
_SIGS = [SIGMA] * _N + [EXACT_SIGMA]


def _km_kernel(idx_cur, idx_nxt, emb_hbm, o1_ref, o2_ref,
               rows0, rows1, sem0, sem1, *, nb, ql, dl, s_steps):
    f1 = 2 * (ql + dl)
    total = nb * f1
    unroll = 16
    n_chunks = total // unroll

    def issue(idx_ref, rows, sem):
        def issue_chunk(c, _):
            base = c * unroll
            for u in range(unroll):
                n = base + u
                t = idx_ref[0, 0, n]
                pltpu.make_async_copy(
                    emb_hbm.at[pl.ds(t, 1), :], rows.at[pl.ds(n, 1), :], sem
                ).start()
            return 0
        jax.lax.fori_loop(0, n_chunks, issue_chunk, 0)

    def wait(rows, sem):
        pltpu.make_async_copy(
            emb_hbm.at[pl.ds(0, total), :], rows.at[pl.ds(0, total), :], sem
        ).wait()

    def compute(rows):
        r = rows[...]
        ss = jnp.sum(r * r, axis=1, keepdims=True)
        rows[...] = r * jax.lax.rsqrt(jnp.maximum(ss, EPS * EPS))

        mm_parts = []
        for pair in range(2):
            for jb in range(nb):
                base = jb * f1 + pair * (ql + dl)
                qn = rows[pl.ds(base, ql), :]
                dn = rows[pl.ds(base + ql, dl), :]
                mm = jax.lax.dot_general(
                    qn, dn, (((1,), (1,)), ((), ())),
                    preferred_element_type=jnp.float32,
                )
                mm_parts.append(mm)
        mm_all = jnp.concatenate(mm_parts, axis=0)

        s_cols = []
        for k in range(KERNEL_NUM):
            c = -0.5 / (_SIGS[k] * _SIGS[k])
            t = mm_all - _MUS[k]
            e = jnp.exp(c * t * t)
            s_cols.append(jnp.sum(e, axis=1, keepdims=True))
        s = jnp.concatenate(s_cols, axis=1)
        l1p = jnp.log1p(s)

        g = 2 * nb
        row_id = jax.lax.broadcasted_iota(jnp.int32, (g * ql, g), 0) // ql
        col_id = jax.lax.broadcasted_iota(jnp.int32, (g * ql, g), 1)
        gmat = (row_id == col_id).astype(jnp.float32)
        km = jax.lax.dot_general(
            gmat, l1p, (((0,), (0,)), ((), ())),
            preferred_element_type=jnp.float32,
            precision=jax.lax.Precision.HIGHEST,
        )
        o1_ref[0] = km[0:nb, :]
        o2_ref[0] = km[nb:2 * nb, :]

    i = pl.program_id(0)
    even = jax.lax.rem(i, 2) == 0

    @pl.when(i == 0)
    def _():
        issue(idx_cur, rows0, sem0)

    @pl.when(jnp.logical_and(i + 1 < s_steps, even))
    def _():
        issue(idx_nxt, rows1, sem1)

    @pl.when(jnp.logical_and(i + 1 < s_steps, jnp.logical_not(even)))
    def _():
        issue(idx_nxt, rows0, sem0)

    @pl.when(even)
    def _():
        wait(rows0, sem0)
        compute(rows0)

    @pl.when(jnp.logical_not(even))
    def _():
        wait(rows1, sem1)
        compute(rows1)


def _mlp_kernel(km1_ref, km2_ref, w0_ref, b0_ref, w1_ref, b1_ref,
                w2_ref, b2_ref, o_ref):
    def head(x):
        h = jax.lax.dot_general(
            x, w0_ref[...], (((1,), (1,)), ((), ())),
            preferred_element_type=jnp.float32,
        ) + b0_ref[...]
        h = jnp.maximum(h, 0.0)
        h = jax.lax.dot_general(
            h, w1_ref[...], (((1,), (1,)), ((), ())),
            preferred_element_type=jnp.float32,
        ) + b1_ref[...]
        h = jnp.maximum(h, 0.0)
        l = jax.lax.dot_general(
            h, w2_ref[...], (((1,), (1,)), ((), ())),
            preferred_element_type=jnp.float32,
        )
        return l[:, 0:1] + b2_ref[0, 0]

    diff = head(km1_ref[...]) - head(km2_ref[...])
    o_ref[...] = 1.0 / (1.0 + jnp.exp(-diff))


def kernel(query_1, document_1, query_2, document_2,
           emb_weight, W0, b0, W1, b1, W2, b2):
    B, QL = query_1.shape
    _, DL = document_1.shape
    V, D = emb_weight.shape

    nb = 8
    while B % nb:
        nb //= 2
    S = B // nb
    f1 = 2 * (QL + DL)

    idx = jnp.concatenate(
        [query_1, document_1, query_2, document_2], axis=1
    ).astype(jnp.int32)
    idx3 = idx.reshape(S, 1, nb * f1)

    km_body = functools.partial(_km_kernel, nb=nb, ql=QL, dl=DL, s_steps=S)
    km1, km2 = pl.pallas_call(
        km_body,
        grid=(S,),
        in_specs=[
            pl.BlockSpec((1, 1, nb * f1), lambda i: (i, 0, 0),
                         memory_space=pltpu.SMEM),
            pl.BlockSpec((1, 1, nb * f1),
                         lambda i: (jnp.minimum(i + 1, S - 1), 0, 0),
                         memory_space=pltpu.SMEM),
            pl.BlockSpec(memory_space=pl.ANY),
        ],
        out_specs=[
            pl.BlockSpec((1, nb, KERNEL_NUM), lambda i: (i, 0, 0)),
            pl.BlockSpec((1, nb, KERNEL_NUM), lambda i: (i, 0, 0)),
        ],
        out_shape=[
            jax.ShapeDtypeStruct((S, nb, KERNEL_NUM), jnp.float32),
            jax.ShapeDtypeStruct((S, nb, KERNEL_NUM), jnp.float32),
        ],
        scratch_shapes=[
            pltpu.VMEM((nb * f1, D), jnp.float32),
            pltpu.VMEM((nb * f1, D), jnp.float32),
            pltpu.SemaphoreType.DMA,
            pltpu.SemaphoreType.DMA,
        ],
        compiler_params=pltpu.CompilerParams(
            dimension_semantics=("arbitrary",),
            disable_bounds_checks=True,
        ),
    )(idx3, idx3, emb_weight)

    out = pl.pallas_call(
        _mlp_kernel,
        out_shape=jax.ShapeDtypeStruct((B, 1), jnp.float32),
    )(
        km1.reshape(B, KERNEL_NUM),
        km2.reshape(B, KERNEL_NUM),
        W0, b0.reshape(1, -1), W1, b1.reshape(1, -1),
        jnp.pad(W2, ((0, 7), (0, 0))), b2.reshape(1, -1),
    )
    return out
